# baseline (speedup 1.0000x reference)
import functools

import jax
import jax.numpy as jnp
from jax import lax
from jax.experimental import pallas as pl
from jax.experimental.pallas import tpu as pltpu
from jax.experimental.pallas import tpu_sc as plsc

_CHUNK = 128


def _make_gather(B, D):
  info = plsc.get_sparse_core_info()
  NC, NS = info.num_cores, info.num_subcores
  NW = NC * NS
  b_per_w = B // NW
  n_chunks = b_per_w // _CHUNK
  mesh = plsc.VectorSubcoreMesh(core_axis_name="c", subcore_axis_name="s")

  @functools.partial(
      pl.kernel,
      mesh=mesh,
      out_type=jax.ShapeDtypeStruct((B, D), jnp.float32),
      scratch_types=[
          pltpu.VMEM((n_chunks, _CHUNK), jnp.int32),
          pltpu.VMEM((n_chunks, _CHUNK, D), jnp.float32),
          pltpu.SemaphoreType.DMA,
      ],
      compiler_params=pltpu.CompilerParams(use_tc_tiling_on_sc=False),
  )
  def gather_kernel(idx_hbm, table_hbm, out_hbm, idx_v, rows_v, sem):
    wid = lax.axis_index("s") * NC + lax.axis_index("c")
    base = wid * n_chunks
    pltpu.sync_copy(idx_hbm.at[pl.ds(base, n_chunks)], idx_v)
    copies = [
        pltpu.async_copy(table_hbm.at[idx_v.at[j]], rows_v.at[j], sem)
        for j in range(n_chunks)
    ]
    for c in copies:
      c.wait()
    for j in range(n_chunks):
      pltpu.sync_copy(
          rows_v.at[j], out_hbm.at[pl.ds((base + j) * _CHUNK, _CHUNK)]
      )

  return gather_kernel


def kernel(inputs, w):
  B = inputs.shape[0]
  D = w.shape[1]
  idx2d = inputs.astype(jnp.int32).reshape(B // _CHUNK, _CHUNK)
  out = _make_gather(B, D)(idx2d, w)
  return out[:, :, None]


# speedup vs baseline: 5.1276x; 5.1276x over previous
import functools

import jax
import jax.numpy as jnp
from jax import lax
from jax.experimental import pallas as pl
from jax.experimental.pallas import tpu as pltpu
from jax.experimental.pallas import tpu_sc as plsc

_G = 16


def _make_gather(B, D, N):
  info = plsc.get_sparse_core_info()
  NC, NS = info.num_cores, info.num_subcores
  NW = NC * NS
  b_per_w = B // NW
  n_groups = b_per_w // _G
  mesh = plsc.VectorSubcoreMesh(core_axis_name="c", subcore_axis_name="s")

  @functools.partial(
      pl.kernel,
      mesh=mesh,
      out_type=jax.ShapeDtypeStruct((D, B), jnp.float32),
      scratch_types=[
          pltpu.VMEM((b_per_w,), jnp.int32),
          pltpu.VMEM((_G, D, 128), jnp.float32),
          pltpu.VMEM((D, b_per_w), jnp.float32),
          pltpu.SemaphoreType.DMA,
      ],
      compiler_params=pltpu.CompilerParams(needs_layout_passes=False),
  )
  def gather_kernel(idx_hbm, table_hbm, out_hbm, idx_v, tiles_v, cols_v, sem):
    wid = lax.axis_index("s") * NC + lax.axis_index("c")
    base = wid * b_per_w
    pltpu.sync_copy(idx_hbm.at[pl.ds(base, b_per_w)], idx_v)

    evec = lax.iota(jnp.int32, _G)

    def group_body(g, carry):
      col0_vec = lax.shift_right_logical(idx_v[pl.ds(g * _G, _G)], 7) * 128
      for e in range(_G):
        col0 = pl.multiple_of(col0_vec[e], 128)
        pltpu.async_copy(
            table_hbm.at[:, pl.ds(col0, 128)], tiles_v.at[e], sem
        )
      for e in range(_G):
        pltpu.make_async_copy(
            table_hbm.at[:, pl.ds(0, 128)], tiles_v.at[e], sem
        ).wait()
      jvec = idx_v[pl.ds(g * _G, _G)] & 127
      bvec = g * _G + evec
      for d in range(D):
        dvec = jnp.full((_G,), d, jnp.int32)
        val = plsc.load_gather(tiles_v, [evec, dvec, jvec])
        plsc.store_scatter(cols_v, [dvec, bvec], val)
      return carry

    lax.fori_loop(0, n_groups, group_body, 0)
    pltpu.sync_copy(cols_v, out_hbm.at[:, pl.ds(base, b_per_w)])

  return gather_kernel


def kernel(inputs, w):
  B = inputs.shape[0]
  N, D = w.shape
  idx = inputs.astype(jnp.int32)
  out_t = _make_gather(B, D, N)(idx, w.T)
  return out_t.T[:, :, None]


# speedup vs baseline: 6.8771x; 1.3412x over previous
import functools

import jax
import jax.numpy as jnp
from jax import lax
from jax.experimental import pallas as pl
from jax.experimental.pallas import tpu as pltpu
from jax.experimental.pallas import tpu_sc as plsc

_G = 16


def _make_gather(B, D, N):
  info = plsc.get_sparse_core_info()
  NC, NS = info.num_cores, info.num_subcores
  NW = NC * NS
  b_per_w = B // NW
  n_groups = b_per_w // _G
  mesh = plsc.VectorSubcoreMesh(core_axis_name="c", subcore_axis_name="s")

  @functools.partial(
      pl.kernel,
      mesh=mesh,
      out_type=jax.ShapeDtypeStruct((D, B), jnp.float32),
      scratch_types=[
          pltpu.VMEM((b_per_w,), jnp.int32),
          pltpu.VMEM((2, _G, D, 128), jnp.float32),
          pltpu.VMEM((D, b_per_w), jnp.float32),
          pltpu.SemaphoreType.DMA,
      ],
      compiler_params=pltpu.CompilerParams(needs_layout_passes=False),
  )
  def gather_kernel(idx_hbm, table_hbm, out_hbm, idx_v, tiles_v, cols_v, sem):
    wid = lax.axis_index("s") * NC + lax.axis_index("c")
    base = wid * b_per_w
    pltpu.sync_copy(idx_hbm.at[pl.ds(base, b_per_w)], idx_v)

    evec = lax.iota(jnp.int32, _G)

    def fire(g, buf):
      col0_vec = lax.shift_right_logical(idx_v[pl.ds(g * _G, _G)], 7) * 128
      for e in range(_G):
        col0 = pl.multiple_of(col0_vec[e], 128)
        pltpu.async_copy(
            table_hbm.at[:, pl.ds(col0, 128)], tiles_v.at[buf, e], sem
        )

    def drain(buf):
      for e in range(_G):
        pltpu.make_async_copy(
            table_hbm.at[:, pl.ds(0, 128)], tiles_v.at[buf, e], sem
        ).wait()

    def extract(g, buf):
      jvec = idx_v[pl.ds(g * _G, _G)] & 127
      bvec = g * _G + evec
      for d in range(D):
        dvec = jnp.full((_G,), d, jnp.int32)
        val = plsc.load_gather(tiles_v.at[buf], [evec, dvec, jvec])
        plsc.store_scatter(cols_v, [dvec, bvec], val)

    fire(0, 0)

    def pair_body(p, carry):
      g0 = p * 2
      fire(g0 + 1, 1)
      drain(0)
      extract(g0, 0)

      @pl.when(g0 + 2 < n_groups)
      def _():
        fire(g0 + 2, 0)

      drain(1)
      extract(g0 + 1, 1)
      return carry

    lax.fori_loop(0, n_groups // 2, pair_body, 0)
    pltpu.sync_copy(cols_v, out_hbm.at[:, pl.ds(base, b_per_w)])

  return gather_kernel


def kernel(inputs, w):
  B = inputs.shape[0]
  N, D = w.shape
  idx = inputs.astype(jnp.int32)
  out_t = _make_gather(B, D, N)(idx, w.T)
  return out_t.T[:, :, None]
